# baseline (speedup 1.0000x reference)
import functools

import jax
import jax.numpy as jnp
from jax import lax
from jax.experimental import pallas as pl
from jax.experimental.pallas import tpu as pltpu
from jax.experimental.pallas import tpu_sc as plsc

B = 16384
V = 50000
NC = 2
NS = 16
NW = NC * NS
SLICE = B // NW
DG = 32
DP = 128

TAG_SPECS = [
    ("rating", 1, 10, 256),
    ("category", 2, 10, 128),
    ("fandom", 5, 20, 32),
    ("relationship", 3, 20, 64),
    ("character", 5, 20, 32),
    ("freeform", 10, 20, 16),
]


def _rsqrt_newton(x):
    bits = lax.bitcast_convert_type(x, jnp.int32)
    y = lax.bitcast_convert_type(
        jnp.int32(0x5F3759DF) - lax.shift_right_logical(bits, 1), jnp.float32)
    for _ in range(3):
        y = y * (1.5 - 0.5 * x * y * y)
    return y


def _sc_bags(idx_r, idx_c, idx_f, idx_rel, idx_ch, idx_fr,
             tab_r, tab_c, tab_f, tab_rel, tab_ch, tab_fr,
             out_r, out_c, out_f, out_rel, out_ch, out_fr,
             ib_r, ib_c, ib_f, ib_rel, ib_ch, ib_fr,
             rows_a, rows_b, out_v,
             isem, osem, gsem_a, gsem_b):
    wid = lax.axis_index("s") * NC + lax.axis_index("c")
    lanes = lax.broadcasted_iota(jnp.int32, (16,), 0)

    idxs = [idx_r, idx_c, idx_f, idx_rel, idx_ch, idx_fr]
    tabs = [tab_r, tab_c, tab_f, tab_rel, tab_ch, tab_fr]
    outs = [out_r, out_c, out_f, out_rel, out_ch, out_fr]
    ibufs = [ib_r, ib_c, ib_f, ib_rel, ib_ch, ib_fr]
    rows2 = [rows_a, rows_b]
    gsems = [gsem_a, gsem_b]

    idescs = [
        pltpu.async_copy(idx_hbm.at[wid], ibuf, isem)
        for idx_hbm, ibuf in zip(idxs, ibufs)
    ]
    for dd in idescs:
        dd.wait()

    prev_out = None
    for ti, (tag, L, d, S) in enumerate(TAG_SPECS):
        tab = tabs[ti]
        ibuf = ibufs[ti]
        nsub = SLICE // S
        ndesc = S * L // DG
        npair = nsub // 2

        def fire(sub_t, par, ndesc=ndesc, tab=tab, ibuf=ibuf):
            rows = rows2[par]
            for g in range(ndesc):
                dg = sub_t * ndesc + g
                pltpu.async_copy(
                    tab.at[ibuf.at[dg // 4, pl.ds((dg % 4) * DG, DG)]],
                    rows.at[pl.ds(g * DG, DG)], gsems[par])

        def drain(par, nrows=S * L, tab=tab):
            pltpu.make_async_copy(
                tab.at[pl.ds(0, nrows)],
                rows2[par].at[pl.ds(0, nrows)], gsems[par]).wait()

        def compute(sub_t, par, L=L, d=d, S=S):
            rows = rows2[par]

            def c16_body(c, carry):
                b_loc = sub_t * S + c * 16
                row_base = (c * 16 + lanes) * L
                accs = [jnp.zeros((16,), jnp.float32) for _ in range(d)]
                for j in range(L):
                    rowv = row_base + j
                    xs = [
                        plsc.load_gather(
                            rows, [rowv, jnp.full((16,), k, jnp.int32)])
                        for k in range(d)
                    ]
                    nsq = xs[0] * xs[0]
                    for k in range(1, d):
                        nsq = nsq + xs[k] * xs[k]
                    scale = jnp.minimum(
                        _rsqrt_newton(jnp.maximum(nsq, 1e-14)), 1.0)
                    for k in range(d):
                        accs[k] = accs[k] + xs[k] * scale
                obase = (b_loc + lanes) * d
                for k in range(d):
                    oflat = obase + k
                    plsc.store_scatter(
                        out_v,
                        [lax.shift_right_logical(oflat, 7),
                         lax.bitwise_and(oflat, 127)],
                        accs[k])
                return carry

            lax.fori_loop(0, S // 16, c16_body, 0)

        fire(0, 0)

        def pair_body(p, carry, fire=fire, drain=drain, compute=compute,
                      npair=npair, ti=ti):
            sub_a = p * 2
            fire(sub_a + 1, 1)
            drain(0)
            compute(sub_a, 0)

            @pl.when(p != npair - 1)
            def _():
                fire(sub_a + 2, 0)

            drain(1)
            compute(sub_a + 1, 1)
            return carry

        if prev_out is not None:
            pltpu.make_async_copy(*prev_out, osem).wait()
        lax.fori_loop(0, npair, pair_body, 0)
        nrow = SLICE * d // 128
        src = out_v.at[pl.ds(0, nrow)]
        dst = outs[ti].at[wid]
        pltpu.async_copy(src, dst, osem)
        prev_out = (src, dst)
    pltpu.make_async_copy(*prev_out, osem).wait()


_sc_call = functools.partial(
    pl.kernel,
    out_type=[jax.ShapeDtypeStruct((NW, SLICE * d // 128, 128), jnp.float32)
              for (_, _, d, _) in TAG_SPECS],
    mesh=plsc.VectorSubcoreMesh(core_axis_name="c", subcore_axis_name="s",
                                num_cores=NC, num_subcores=NS),
    scratch_types=(
        [pltpu.VMEM((SLICE * L // 128, 128), jnp.int32)
         for (_, L, _, _) in TAG_SPECS]
        + [
            pltpu.VMEM((256, DP), jnp.float32),
            pltpu.VMEM((256, DP), jnp.float32),
            pltpu.VMEM((SLICE * 20 // 128, 128), jnp.float32),
            pltpu.SemaphoreType.DMA,
            pltpu.SemaphoreType.DMA,
            pltpu.SemaphoreType.DMA,
            pltpu.SemaphoreType.DMA,
        ]
    ),
    compiler_params=pltpu.CompilerParams(needs_layout_passes=False,
                                         use_tc_tiling_on_sc=True),
)(_sc_bags)


def _tc_body(dense_ref, r_ref, c_ref, f_ref, rel_ref, ch_ref, fr_ref,
             Wd_ref, bd_ref, W1_ref, b1_ref, W2_ref, b2_ref,
             z_ref, pre_ref, de_ref):
    de = jnp.dot(dense_ref[...], Wd_ref[...],
                 preferred_element_type=jnp.float32) + bd_ref[...]
    r = r_ref[...]
    c = c_ref[...]
    f = f_ref[...]
    rel = rel_ref[...]
    ch = ch_ref[...]
    fr = fr_ref[...]
    basic = jnp.concatenate([r, c], axis=-1)

    def dot(a, b):
        return jnp.sum(a * b, axis=-1, keepdims=True)

    pre = jnp.concatenate([
        de, r, c, f, rel, ch, fr,
        dot(de, basic), dot(de, f), dot(de, rel), dot(de, ch), dot(de, fr),
        dot(basic, f), dot(basic, rel), dot(basic, ch), dot(basic, fr),
        dot(f, rel), dot(f, ch), dot(f, fr),
        dot(rel, ch), dot(rel, fr),
        dot(ch, fr)
    ], axis=1)
    h = jnp.dot(pre, W1_ref[...], preferred_element_type=jnp.float32) + b1_ref[...]
    h = jnp.where(h > 0, h, 0.01 * h)
    z_ref[...] = jnp.dot(h, W2_ref[...],
                         preferred_element_type=jnp.float32) + b2_ref[...]
    pre_ref[...] = pre
    de_ref[...] = de


def _tc_call(dense, r, c, f, rel, ch, fr, Wd, bd, W1, b1, W2, b2):
    BM = 2048
    grid = B // BM

    def rows(d):
        return pl.BlockSpec((BM, d), lambda i: (i, 0))

    def whole(shape):
        return pl.BlockSpec(shape, lambda i: (0, 0))

    return pl.pallas_call(
        _tc_body,
        grid=(grid,),
        in_specs=[
            rows(16), rows(10), rows(10), rows(20), rows(20), rows(20), rows(20),
            whole((16, 20)), whole((1, 20)),
            whole((135, 64)), whole((1, 64)),
            whole((64, 128)), whole((1, 128)),
        ],
        out_specs=[rows(128), rows(135), rows(20)],
        out_shape=[
            jax.ShapeDtypeStruct((B, 128), jnp.float32),
            jax.ShapeDtypeStruct((B, 135), jnp.float32),
            jax.ShapeDtypeStruct((B, 20), jnp.float32),
        ],
    )(dense, r, c, f, rel, ch, fr, Wd, bd, W1, b1, W2, b2)


@jax.jit
def kernel(dense, idx_rating, idx_category, idx_fandom, idx_relationship,
           idx_character, idx_freeform,
           emb_rating, emb_category, emb_fandom, emb_relationship,
           emb_character, emb_freeform,
           Wd, bd, W1, b1, W2, b2):
    idxs = [idx_rating, idx_category, idx_fandom, idx_relationship,
            idx_character, idx_freeform]
    idx_3d = [i.reshape(NW, -1, 128) for i in idxs]
    tabs = [emb_rating, emb_category, emb_fandom, emb_relationship,
            emb_character, emb_freeform]
    tabs_pad = [
        jnp.pad(t, ((0, 0), (0, DP - d)))
        for t, (_, _, d, _) in zip(tabs, TAG_SPECS)
    ]
    bags_3d = _sc_call(*idx_3d, *tabs_pad)
    bags = [b.reshape(B, d) for b, (_, _, d, _) in zip(bags_3d, TAG_SPECS)]
    z, pre_over, de = _tc_call(
        dense, *bags, Wd, bd.reshape(1, -1), W1, b1.reshape(1, -1),
        W2, b2.reshape(1, -1))
    return (z, pre_over, de)
